# baseline (speedup 1.0000x reference)
import jax
import jax.numpy as jnp
from jax.experimental import pallas as pl

N = 4096
NFEAT = 128
NHID = 128
XW = 2 * NHID
BLK = 2048
NBLK = N // BLK


def _mega_body(feat_ref, wft_ref, bf_ref, a2_ref, w0t_ref, b0_ref,
               w1t_ref, b1_ref, o_ref):
    feat = feat_ref[...]
    h = jnp.dot(feat, wft_ref[...], preferred_element_type=jnp.float32)
    h = h + bf_ref[...]
    f = jnp.dot(h, a2_ref[...], preferred_element_type=jnp.float32)
    f1 = f[:, 0:1]
    f2 = f[:, 1:2].T
    c1 = jnp.max(f[:, 0])
    c2 = jnp.max(f[:, 1])
    x0 = jnp.dot(feat, w0t_ref[...], preferred_element_type=jnp.float32)
    x0 = (x0 + b0_ref[...]).astype(jnp.bfloat16)
    ones = jnp.ones((N, NHID), jnp.bfloat16)
    x_ext = jnp.concatenate([x0, ones], axis=1)

    b1v = jnp.exp(f2 - c2).astype(jnp.bfloat16)
    b2v = jnp.exp(0.2 * (f2 - c2) - 0.8 * (c1 + c2)).astype(jnp.bfloat16)
    r_all = jnp.exp(0.8 * (f1 - c1)).astype(jnp.bfloat16)

    y_blocks = []
    for b in range(NBLK):
        r = r_all[b * BLK:(b + 1) * BLK, :]
        e = jnp.maximum(r * b1v, b2v)
        p = jnp.dot(e, x_ext, preferred_element_type=jnp.float32)
        y = p[:, :NHID] / p[:, NHID:NHID + 1]
        y = jnp.where(y > 0.0, y, jnp.exp(y) - 1.0)
        y_blocks.append(y.astype(jnp.bfloat16))
    y_ext = jnp.concatenate([jnp.concatenate(y_blocks, axis=0), ones], axis=1)

    for b in range(NBLK):
        r = r_all[b * BLK:(b + 1) * BLK, :]
        e = jnp.maximum(r * b1v, b2v)
        p = jnp.dot(e, y_ext, preferred_element_type=jnp.float32)
        y = p[:, :NHID] / p[:, NHID:NHID + 1]
        z = jnp.dot(y, w1t_ref[...], preferred_element_type=jnp.float32)
        z = z + b1_ref[...]
        o_ref[b * BLK:(b + 1) * BLK, :] = jnp.where(
            z > 0.0, z, jnp.exp(z) - 1.0)


@jax.jit
def kernel(feat_data, adjs, Wf, bf, a_src, a_dest, W0, b0, W1, b1):
    del adjs
    a2 = jnp.concatenate([a_src, a_dest], axis=1)
    return pl.pallas_call(
        _mega_body,
        out_shape=jax.ShapeDtypeStruct((N, NHID), jnp.float32),
    )(feat_data, Wf.T, bf.reshape(1, NHID), a2, W0.T, b0.reshape(1, NHID),
      W1.T, b1.reshape(1, NHID))


# speedup vs baseline: 1.0095x; 1.0095x over previous
import jax
import jax.numpy as jnp
from jax.experimental import pallas as pl

N = 4096
NFEAT = 128
NHID = 128
XW = 2 * NHID
BLK = 2048
NBLK = N // BLK


def _mega_body(feat_ref, wft_ref, bf_ref, a2_ref, w0t_ref, b0_ref,
               w1t_ref, b1_ref, o_ref):
    feat = feat_ref[...]
    h = jnp.dot(feat, wft_ref[...], preferred_element_type=jnp.float32)
    h = h + bf_ref[...]
    f = jnp.dot(h, a2_ref[...], preferred_element_type=jnp.float32)
    f1 = f[:, 0:1]
    f2 = f[:, 1:2].T
    c1 = jnp.max(f[:, 0])
    c2 = jnp.max(f[:, 1])
    x0 = jnp.dot(feat, w0t_ref[...], preferred_element_type=jnp.float32)
    x0 = (x0 + b0_ref[...]).astype(jnp.bfloat16)
    ones = jnp.ones((N, NHID), jnp.bfloat16)
    x_ext = jnp.concatenate([x0, ones], axis=1)

    b1v = jnp.exp(f2 - c2).astype(jnp.bfloat16)
    b2v = jnp.exp(0.2 * (f2 - c2) - 0.8 * (c1 + c2)).astype(jnp.bfloat16)
    r_all = jnp.exp(0.8 * (f1 - c1)).astype(jnp.bfloat16)

    y_blocks = []
    e_blocks = []
    s_blocks = []
    for b in range(NBLK):
        r = r_all[b * BLK:(b + 1) * BLK, :]
        e = jnp.maximum(r * b1v, b2v)
        e_blocks.append(e)
        p = jnp.dot(e, x_ext, preferred_element_type=jnp.float32)
        s = p[:, NHID:NHID + 1]
        s_blocks.append(s)
        y = p[:, :NHID] / s
        y = jnp.where(y > 0.0, y, jnp.exp(y) - 1.0)
        y_blocks.append(y.astype(jnp.bfloat16))
    y_all = jnp.concatenate(y_blocks, axis=0)

    for b in range(NBLK):
        p = jnp.dot(e_blocks[b], y_all, preferred_element_type=jnp.float32)
        y = p / s_blocks[b]
        z = jnp.dot(y, w1t_ref[...], preferred_element_type=jnp.float32)
        z = z + b1_ref[...]
        o_ref[b * BLK:(b + 1) * BLK, :] = jnp.where(
            z > 0.0, z, jnp.exp(z) - 1.0)


@jax.jit
def kernel(feat_data, adjs, Wf, bf, a_src, a_dest, W0, b0, W1, b1):
    del adjs
    a2 = jnp.concatenate([a_src, a_dest], axis=1)
    return pl.pallas_call(
        _mega_body,
        out_shape=jax.ShapeDtypeStruct((N, NHID), jnp.float32),
    )(feat_data, Wf.T, bf.reshape(1, NHID), a2, W0.T, b0.reshape(1, NHID),
      W1.T, b1.reshape(1, NHID))


# speedup vs baseline: 1.0214x; 1.0117x over previous
import jax
import jax.numpy as jnp
from jax.experimental import pallas as pl

N = 4096
NFEAT = 128
NHID = 128
XW = 2 * NHID
BLK = 1024
NBLK = N // BLK


def _prep_body(feat_ref, wft_ref, bf_ref, a2_ref, w0t_ref, b0_ref,
               f1_ref, f2r_ref, c_ref, x0_ref):
    feat = feat_ref[...]
    h = jnp.dot(feat, wft_ref[...], preferred_element_type=jnp.float32)
    h = h + bf_ref[...]
    f = jnp.dot(h, a2_ref[...], preferred_element_type=jnp.float32)
    f1_ref[...] = f[:, 0:1]
    f2r_ref[...] = f[:, 1:2].T
    c_ref[...] = jnp.max(f, axis=0, keepdims=True)
    x0 = jnp.dot(feat, w0t_ref[...], preferred_element_type=jnp.float32)
    x0 = (x0 + b0_ref[...]).astype(jnp.bfloat16)
    ones = jnp.ones((N, NHID), jnp.bfloat16)
    x0_ref[...] = jnp.concatenate([x0, ones], axis=1)


def _attn_e(f1, f2, c1, c2):
    r = jnp.exp(0.8 * (f1 - c1)).astype(jnp.bfloat16)
    b1v = jnp.exp(f2 - c2).astype(jnp.bfloat16)
    b2v = jnp.exp(0.2 * (f2 - c2) - 0.8 * (c1 + c2)).astype(jnp.bfloat16)
    return jnp.maximum(r * b1v, b2v)


def _attn1_body(f1_ref, f2_ref, c_ref, x_ref, o_ref):
    e = _attn_e(f1_ref[...], f2_ref[...], c_ref[0, 0], c_ref[0, 1])
    p = jnp.dot(e, x_ref[...], preferred_element_type=jnp.float32)
    y = p[:, :NHID] / p[:, NHID:NHID + 1]
    y = jnp.where(y > 0.0, y, jnp.exp(y) - 1.0)
    ones = jnp.ones((y.shape[0], NHID), jnp.bfloat16)
    o_ref[...] = jnp.concatenate([y.astype(jnp.bfloat16), ones], axis=1)


def _attn2_body(f1_ref, f2_ref, c_ref, x_ref, w1t_ref, b1_ref, o_ref):
    e = _attn_e(f1_ref[...], f2_ref[...], c_ref[0, 0], c_ref[0, 1])
    p = jnp.dot(e, x_ref[...], preferred_element_type=jnp.float32)
    y = p[:, :NHID] / p[:, NHID:NHID + 1]
    z = jnp.dot(y, w1t_ref[...], preferred_element_type=jnp.float32)
    z = z + b1_ref[...]
    o_ref[...] = jnp.where(z > 0.0, z, jnp.exp(z) - 1.0)


@jax.jit
def kernel(feat_data, adjs, Wf, bf, a_src, a_dest, W0, b0, W1, b1):
    del adjs
    a2 = jnp.concatenate([a_src, a_dest], axis=1)

    f1c, f2r, c, x0 = pl.pallas_call(
        _prep_body,
        out_shape=[
            jax.ShapeDtypeStruct((N, 1), jnp.float32),
            jax.ShapeDtypeStruct((1, N), jnp.float32),
            jax.ShapeDtypeStruct((1, 2), jnp.float32),
            jax.ShapeDtypeStruct((N, XW), jnp.bfloat16),
        ],
    )(feat_data, Wf.T, bf.reshape(1, NHID), a2, W0.T, b0.reshape(1, NHID))

    attn_specs = [
        pl.BlockSpec((BLK, 1), lambda i: (i, 0)),
        pl.BlockSpec((1, N), lambda i: (0, 0)),
        pl.BlockSpec((1, 2), lambda i: (0, 0)),
        pl.BlockSpec((N, XW), lambda i: (0, 0)),
    ]

    y0 = pl.pallas_call(
        _attn1_body,
        grid=(NBLK,),
        in_specs=attn_specs,
        out_specs=pl.BlockSpec((BLK, XW), lambda i: (i, 0)),
        out_shape=jax.ShapeDtypeStruct((N, XW), jnp.bfloat16),
    )(f1c, f2r, c, x0)

    out = pl.pallas_call(
        _attn2_body,
        grid=(NBLK,),
        in_specs=attn_specs + [
            pl.BlockSpec((NHID, NHID), lambda i: (0, 0)),
            pl.BlockSpec((1, NHID), lambda i: (0, 0)),
        ],
        out_specs=pl.BlockSpec((BLK, NHID), lambda i: (i, 0)),
        out_shape=jax.ShapeDtypeStruct((N, NHID), jnp.float32),
    )(f1c, f2r, c, y0, W1.T, b1.reshape(1, NHID))

    return out
